# baseline (speedup 1.0000x reference)
import jax
import jax.numpy as jnp
from jax.experimental import pallas as pl
from jax.experimental.pallas import tpu as pltpu

_LP = 8


def _conv9(src_flat, w_ref, s_ref, t_ref, wp, nq, tm, out_cb, out_f32):
    offs = [dy * wp + dx + _LP - 1 for dy in range(3) for dx in range(3)]
    npairs = nq // (2 * tm)
    pending = None
    for p in range(npairs):
        addr = (p % 3) * (tm // 4)
        for m in range(2):
            pltpu.matmul_push_rhs(w_ref[0], 0, m)
        for t in range(9):
            for m in range(2):
                if t + 1 < 9:
                    pltpu.matmul_push_rhs(w_ref[t + 1], (t + 1) % 2, m)
                m0 = (2 * p + m) * tm
                pltpu.matmul_acc_lhs(addr, src_flat[m0 + offs[t]:
                                                    m0 + offs[t] + tm],
                                     m, load_staged_rhs=t % 2)
        if pending is not None:
            _drain(pending, w_ref, s_ref, t_ref, tm, out_cb, out_f32)
        pending = (p, (p % 3) * (tm // 4))
    _drain(pending, w_ref, s_ref, t_ref, tm, out_cb, out_f32)


def _drain(pending, w_ref, s_ref, t_ref, tm, out_cb, out_f32):
    p, addr = pending
    for m in range(2):
        acc = pltpu.matmul_pop(addr, (tm, 256), jnp.float32, m)
        r = jnp.maximum(acc * s_ref[...] + t_ref[...], 0.0)
        if not out_f32:
            r = r.astype(jnp.bfloat16)
        out_cb(2 * p + m, r)


def _up_dc_kernel(x1_ref, x2r_ref, wup_ref, b4_ref, w1_ref, s1_ref, t1_ref,
                  w2_ref, s2_ref, t2_ref, o_ref, s5_ref, p2_ref):
    f32 = jnp.float32
    bf16 = jnp.bfloat16
    h1 = x2r_ref.shape[1]
    ch = x2r_ref.shape[4]
    cout = w1_ref.shape[2]
    h2 = 2 * h1
    wp = s5_ref.shape[2]
    w2 = wp - _LP
    nq = h2 * wp
    tm = 4 * wp

    s5_ref[:, :, 0:_LP, :] = jnp.zeros(
        (s5_ref.shape[0], 2, _LP, 2 * ch), bf16)
    zrow = jnp.zeros((w2, 2 * ch), bf16)
    s5_ref[0, 0, _LP:, :] = zrow
    s5_ref[h1, 1, _LP:, :] = zrow
    s5_ref[h1 + 1, 0, _LP:, :] = zrow
    s5_ref[0:h1, 1, _LP:, 0:ch] = x2r_ref[0, :, 0]
    s5_ref[1:1 + h1, 0, _LP:, 0:ch] = x2r_ref[0, :, 1]

    x1v = x1_ref[0]
    mh = x1v.shape[0] // 2
    for m in range(2):
        pltpu.matmul_push_rhs(wup_ref[:, 0:256], 0, m)
        pltpu.matmul_acc_lhs(0, x1v[m * mh:(m + 1) * mh], m,
                             load_staged_rhs=0)
        pltpu.matmul_push_rhs(wup_ref[:, 256:512], 1, m)
        pltpu.matmul_acc_lhs(128, x1v[m * mh:(m + 1) * mh], m,
                             load_staged_rhs=1)
    hh = h1 // 2
    for m in range(2):
        ye = (pltpu.matmul_pop(0, (mh, 256), f32, m)
              + b4_ref[:, 0:256]).astype(bf16).reshape(hh, w2, ch)
        yo = (pltpu.matmul_pop(128, (mh, 256), f32, m)
              + b4_ref[:, 256:512]).astype(bf16).reshape(hh, w2, ch)
        s5_ref[m * hh:(m + 1) * hh, 1, _LP:, ch:2 * ch] = ye
        s5_ref[1 + m * hh:1 + (m + 1) * hh, 0, _LP:, ch:2 * ch] = yo

    sf = s5_ref[...].reshape(s5_ref.shape[0] * 2 * wp, 2 * ch)
    zc = jnp.zeros((_LP, cout), bf16)
    p2_ref[0, 0:_LP, :] = zc
    p2_ref[0, _LP:, :] = jnp.zeros((w2, cout), bf16)
    p2_ref[h2 + 1, :, :] = jnp.zeros((wp, cout), bf16)
    p2_ref[h2 + 2, :, :] = jnp.zeros((wp, cout), bf16)

    def store_y1(i, r):
        rb = r.reshape(4, wp, cout)[:, 0:w2, :]
        p2_ref[1 + 4 * i:5 + 4 * i, _LP:, :] = rb
        p2_ref[1 + 4 * i:5 + 4 * i, 0:_LP, :] = jnp.broadcast_to(
            zc.reshape(1, _LP, cout), (4, _LP, cout))

    _conv9(sf, w1_ref, s1_ref, t1_ref, wp, nq, tm, store_y1, False)

    pf = p2_ref[...].reshape(p2_ref.shape[0] * wp, cout)

    def store_z(i, r):
        rb = r.reshape(4, wp, cout)[:, 0:w2, :]
        o_ref[0, 4 * i:4 + 4 * i, :, :] = rb

    _conv9(pf, w2_ref, s2_ref, t2_ref, wp, nq, tm, store_z, True)


def kernel(up_w, up_b, conv1_w, conv1_b, bn1_gamma, bn1_beta, bn1_mean,
           bn1_var, conv2_w, conv2_b, bn2_gamma, bn2_beta, bn2_mean,
           bn2_var, x1, x2):
    f32 = jnp.float32
    bf16 = jnp.bfloat16
    n, cin, h1, w1sp = x1.shape
    ch = cin // 2
    h2, w2 = 2 * h1, 2 * w1sp
    cout = conv1_w.shape[-1]
    wp = _LP + w2
    rows5 = (h2 + 4) // 2

    x1r = jnp.transpose(x1, (0, 2, 3, 1)).astype(bf16).reshape(
        n, h1 * w1sp, cin)
    x2n = jnp.transpose(x2, (0, 2, 3, 1)).astype(bf16)
    x2r = x2n.reshape(n, h1, 2, w2, ch)
    wup = up_w.astype(bf16).reshape(cin, 4 * ch)
    b4 = jnp.tile(up_b.astype(f32), 4).reshape(1, 4 * ch)
    w1r = conv1_w.astype(bf16).reshape(9, cin, cout)
    w2r = conv2_w.astype(bf16).reshape(9, cout, cout)

    def fold(b, g, bt, m, v):
        s = g / jnp.sqrt(v + 1e-5)
        return (s.reshape(1, cout).astype(f32),
                ((b - m) * s + bt).reshape(1, cout).astype(f32))

    s1, t1 = fold(conv1_b, bn1_gamma, bn1_beta, bn1_mean, bn1_var)
    s2, t2 = fold(conv2_b, bn2_gamma, bn2_beta, bn2_mean, bn2_var)

    out = pl.pallas_call(
        _up_dc_kernel,
        out_shape=jax.ShapeDtypeStruct((n, h2, w2, cout), f32),
        grid=(n,),
        in_specs=[
            pl.BlockSpec((1, h1 * w1sp, cin), lambda i: (i, 0, 0)),
            pl.BlockSpec((1, h1, 2, w2, ch), lambda i: (i, 0, 0, 0, 0)),
            pl.BlockSpec((cin, 4 * ch), lambda i: (0, 0)),
            pl.BlockSpec((1, 4 * ch), lambda i: (0, 0)),
            pl.BlockSpec((9, cin, cout), lambda i: (0, 0, 0)),
            pl.BlockSpec((1, cout), lambda i: (0, 0)),
            pl.BlockSpec((1, cout), lambda i: (0, 0)),
            pl.BlockSpec((9, cout, cout), lambda i: (0, 0, 0)),
            pl.BlockSpec((1, cout), lambda i: (0, 0)),
            pl.BlockSpec((1, cout), lambda i: (0, 0)),
        ],
        out_specs=pl.BlockSpec((1, h2, w2, cout), lambda i: (i, 0, 0, 0)),
        scratch_shapes=[
            pltpu.VMEM((rows5, 2, wp, cin), bf16),
            pltpu.VMEM((h2 + 4, wp, cout), bf16),
        ],
        compiler_params=pltpu.CompilerParams(
            dimension_semantics=("parallel",)),
    )(x1r, x2r, wup, b4, w1r, s1, t1, w2r, s2, t2)

    return out


# speedup vs baseline: 1.0923x; 1.0923x over previous
import jax
import jax.numpy as jnp
from jax.experimental import pallas as pl
from jax.experimental.pallas import tpu as pltpu

_LP = 8


def _conv9(src_flat, w_ref, s_ref, t_ref, wp, nq, tm, out_cb, out_f32):
    offs = [dy * wp + dx + _LP - 1 for dy in range(3) for dx in range(3)]
    npairs = nq // (2 * tm)
    pending = None
    for p in range(npairs):
        addr = (p % 3) * (tm // 4)
        for m in range(2):
            pltpu.matmul_push_rhs(w_ref[0], 0, m)
        for t in range(9):
            for m in range(2):
                if t + 1 < 9:
                    pltpu.matmul_push_rhs(w_ref[t + 1], (t + 1) % 2, m)
                m0 = (2 * p + m) * tm
                pltpu.matmul_acc_lhs(addr, src_flat[m0 + offs[t]:
                                                    m0 + offs[t] + tm],
                                     m, load_staged_rhs=t % 2)
        if pending is not None:
            _drain(pending, w_ref, s_ref, t_ref, tm, out_cb, out_f32)
        pending = (p, (p % 3) * (tm // 4))
    _drain(pending, w_ref, s_ref, t_ref, tm, out_cb, out_f32)


def _drain(pending, w_ref, s_ref, t_ref, tm, out_cb, out_f32):
    p, addr = pending
    for m in range(2):
        acc = pltpu.matmul_pop(addr, (tm, 256), jnp.float32, m)
        r = jnp.maximum(acc * s_ref[...] + t_ref[...], 0.0)
        if not out_f32:
            r = r.astype(jnp.bfloat16)
        out_cb(2 * p + m, r)


def _up_dc_kernel(x1_ref, x2r_ref, wup_ref, b4_ref, w1_ref, s1_ref, t1_ref,
                  w2_ref, s2_ref, t2_ref, o_ref, s5_ref, p2_ref):
    f32 = jnp.float32
    bf16 = jnp.bfloat16
    h1 = x2r_ref.shape[1]
    ch = x2r_ref.shape[4]
    cout = w1_ref.shape[2]
    h2 = 2 * h1
    wp = s5_ref.shape[2]
    w2 = wp - _LP
    nq = h2 * wp
    tm = 4 * wp

    s5_ref[:, :, 0:_LP, :] = jnp.zeros(
        (s5_ref.shape[0], 2, _LP, 2 * ch), bf16)
    zrow = jnp.zeros((w2, 2 * ch), bf16)
    s5_ref[0, 0, _LP:, :] = zrow
    s5_ref[h1, 1, _LP:, :] = zrow
    s5_ref[h1 + 1, 0, _LP:, :] = zrow
    s5_ref[0:h1, 1, _LP:, 0:ch] = x2r_ref[0, :, 0].astype(bf16)
    s5_ref[1:1 + h1, 0, _LP:, 0:ch] = x2r_ref[0, :, 1].astype(bf16)

    x1v = x1_ref[0].astype(bf16)
    mh = x1v.shape[0] // 2
    for m in range(2):
        pltpu.matmul_push_rhs(wup_ref[:, 0:256], 0, m)
        pltpu.matmul_acc_lhs(0, x1v[m * mh:(m + 1) * mh], m,
                             load_staged_rhs=0)
        pltpu.matmul_push_rhs(wup_ref[:, 256:512], 1, m)
        pltpu.matmul_acc_lhs(128, x1v[m * mh:(m + 1) * mh], m,
                             load_staged_rhs=1)
    hh = h1 // 2
    for m in range(2):
        ye = (pltpu.matmul_pop(0, (mh, 256), f32, m)
              + b4_ref[:, 0:256]).astype(bf16).reshape(hh, w2, ch)
        yo = (pltpu.matmul_pop(128, (mh, 256), f32, m)
              + b4_ref[:, 256:512]).astype(bf16).reshape(hh, w2, ch)
        s5_ref[m * hh:(m + 1) * hh, 1, _LP:, ch:2 * ch] = ye
        s5_ref[1 + m * hh:1 + (m + 1) * hh, 0, _LP:, ch:2 * ch] = yo

    sf = s5_ref[...].reshape(s5_ref.shape[0] * 2 * wp, 2 * ch)
    zc = jnp.zeros((_LP, cout), bf16)
    p2_ref[0, 0:_LP, :] = zc
    p2_ref[0, _LP:, :] = jnp.zeros((w2, cout), bf16)
    p2_ref[h2 + 1, :, :] = jnp.zeros((wp, cout), bf16)
    p2_ref[h2 + 2, :, :] = jnp.zeros((wp, cout), bf16)

    def store_y1(i, r):
        rb = r.reshape(4, wp, cout)[:, 0:w2, :]
        p2_ref[1 + 4 * i:5 + 4 * i, _LP:, :] = rb
        p2_ref[1 + 4 * i:5 + 4 * i, 0:_LP, :] = jnp.broadcast_to(
            zc.reshape(1, _LP, cout), (4, _LP, cout))

    _conv9(sf, w1_ref, s1_ref, t1_ref, wp, nq, tm, store_y1, False)

    pf = p2_ref[...].reshape(p2_ref.shape[0] * wp, cout)

    def store_z(i, r):
        rb = r.reshape(4, wp, cout)[:, 0:w2, :]
        o_ref[0, 4 * i:4 + 4 * i, :, :] = rb

    _conv9(pf, w2_ref, s2_ref, t2_ref, wp, nq, tm, store_z, True)


def kernel(up_w, up_b, conv1_w, conv1_b, bn1_gamma, bn1_beta, bn1_mean,
           bn1_var, conv2_w, conv2_b, bn2_gamma, bn2_beta, bn2_mean,
           bn2_var, x1, x2):
    f32 = jnp.float32
    bf16 = jnp.bfloat16
    n, cin, h1, w1sp = x1.shape
    ch = cin // 2
    h2, w2 = 2 * h1, 2 * w1sp
    cout = conv1_w.shape[-1]
    wp = _LP + w2
    rows5 = (h2 + 4) // 2

    x1r = jnp.transpose(x1, (0, 2, 3, 1)).reshape(n, h1 * w1sp, cin)
    x2r = jnp.transpose(x2, (0, 2, 3, 1)).reshape(n, h1, 2, w2, ch)
    wup = up_w.astype(bf16).reshape(cin, 4 * ch)
    b4 = jnp.tile(up_b.astype(f32), 4).reshape(1, 4 * ch)
    w1r = conv1_w.astype(bf16).reshape(9, cin, cout)
    w2r = conv2_w.astype(bf16).reshape(9, cout, cout)

    def fold(b, g, bt, m, v):
        s = g / jnp.sqrt(v + 1e-5)
        return (s.reshape(1, cout).astype(f32),
                ((b - m) * s + bt).reshape(1, cout).astype(f32))

    s1, t1 = fold(conv1_b, bn1_gamma, bn1_beta, bn1_mean, bn1_var)
    s2, t2 = fold(conv2_b, bn2_gamma, bn2_beta, bn2_mean, bn2_var)

    out = pl.pallas_call(
        _up_dc_kernel,
        out_shape=jax.ShapeDtypeStruct((n, h2, w2, cout), f32),
        grid=(n,),
        in_specs=[
            pl.BlockSpec((1, h1 * w1sp, cin), lambda i: (i, 0, 0)),
            pl.BlockSpec((1, h1, 2, w2, ch), lambda i: (i, 0, 0, 0, 0)),
            pl.BlockSpec((cin, 4 * ch), lambda i: (0, 0)),
            pl.BlockSpec((1, 4 * ch), lambda i: (0, 0)),
            pl.BlockSpec((9, cin, cout), lambda i: (0, 0, 0)),
            pl.BlockSpec((1, cout), lambda i: (0, 0)),
            pl.BlockSpec((1, cout), lambda i: (0, 0)),
            pl.BlockSpec((9, cout, cout), lambda i: (0, 0, 0)),
            pl.BlockSpec((1, cout), lambda i: (0, 0)),
            pl.BlockSpec((1, cout), lambda i: (0, 0)),
        ],
        out_specs=pl.BlockSpec((1, h2, w2, cout), lambda i: (i, 0, 0, 0)),
        scratch_shapes=[
            pltpu.VMEM((rows5, 2, wp, cin), bf16),
            pltpu.VMEM((h2 + 4, wp, cout), bf16),
        ],
        compiler_params=pltpu.CompilerParams(
            dimension_semantics=("parallel",)),
    )(x1r, x2r, wup, b4, w1r, s1, t1, w2r, s2, t2)

    return jnp.transpose(out, (0, 3, 1, 2))
